# baseline (speedup 1.0000x reference)
import jax
import jax.numpy as jnp
from jax.experimental import pallas as pl

N = 10000
E = 320000
D = 128
DE = 16
H = 4
HID = 128
C1 = HID // H


def _gat_layer(x, src, dst, eattr, p, heads, cper, concat, n):
    xl = (x @ p['Wl'] + p['bl']).reshape(n, heads, cper)
    xr = (x @ p['Wr'] + p['br']).reshape(n, heads, cper)
    ee = (eattr @ p['We']).reshape(-1, heads, cper)
    msg = xl[src]
    ker = jax.nn.leaky_relu(msg + xr[dst] + ee, negative_slope=0.2)
    alpha = (ker * p['att'][None, :, :]).sum(-1)
    amax = jax.ops.segment_max(alpha, dst, num_segments=n)
    amax = jnp.where(jnp.isfinite(amax), amax, 0.0)
    ex = jnp.exp(alpha - amax[dst])
    den = jax.ops.segment_sum(ex, dst, num_segments=n)
    a = ex / (den[dst] + 1e-16)
    out = jax.ops.segment_sum(msg * a[:, :, None], dst, num_segments=n)
    out = out.reshape(n, heads * cper) if concat else out.mean(axis=1)
    return out + p['bias']


def _mlp_kernel(g_ref, w1_ref, b1_ref, w2_ref, b2_ref, out_ref):
    h = jnp.maximum(jnp.sum(g_ref[:].T * w1_ref[:], axis=0, keepdims=True) + b1_ref[:], 0.0)
    s = jnp.sum(h.T * w2_ref[:], axis=0, keepdims=True) + b2_ref[:]
    out_ref[:] = 1.0 / (1.0 + jnp.exp(-s))


def kernel(node_features, edge_attr, epr_scores, question_emb, params, edge_index):
    n = node_features.shape[0]
    src, dst = edge_index[0], edge_index[1]
    ef = jax.nn.relu(epr_scores[:, None] @ params['e1_w'] + params['e1_b'])
    ef = ef @ params['e2_w'] + params['e2_b']
    ea = jnp.concatenate([edge_attr, ef], axis=-1)
    cnt = jax.ops.segment_sum(jnp.ones((src.shape[0],), jnp.float32), dst, num_segments=n)
    mean_e = jax.ops.segment_sum(ea, dst, num_segments=n) / jnp.maximum(cnt, 1.0)[:, None]
    loop = jnp.arange(n, dtype=src.dtype)
    src2 = jnp.concatenate([src, loop])
    dst2 = jnp.concatenate([dst, loop])
    ea2 = jnp.concatenate([ea, mean_e], axis=0)
    x = _gat_layer(node_features, src2, dst2, ea2, params['g0'], H, C1, True, n)
    x = jax.nn.elu(x)
    x = _gat_layer(x, src2, dst2, ea2, params['g1'], H, C1, True, n)
    x = jax.nn.elu(x)
    x = _gat_layer(x, src2, dst2, ea2, params['g2'], H, HID, False, n)
    graph_repr = x.mean(axis=0)
    path_score = pl.pallas_call(
        _mlp_kernel,
        out_shape=jax.ShapeDtypeStruct((1, 1), jnp.float32),
    )(graph_repr[None, :], params['s1_w'], params['s1_b'][None, :],
      params['s2_w'], params['s2_b'][None, :])
    return path_score.reshape(1), x, graph_repr


# speedup vs baseline: 7.4601x; 7.4601x over previous
import functools

import jax
import jax.numpy as jnp
from jax import lax
from jax.experimental import pallas as pl
from jax.experimental.pallas import tpu as pltpu
from jax.experimental.pallas import tpu_sc as plsc

N = 10000
E = 320000
NC, NS, LN = 2, 16, 16
NW = NC * NS
NP = 10240
NP4 = NP * 4
E2 = E + N
EPW = 10368
E2P = NW * EPW


def _mesh():
    return plsc.VectorSubcoreMesh(core_axis_name="c", subcore_axis_name="s")


def _wid():
    return lax.axis_index("s") * NC + lax.axis_index("c")


def _sc_gather_body(table, idx, out, idxb, rowsb, sem, *, cb):
    wid = _wid()

    def chunk(i, c):
        eb = wid * EPW + i * cb
        pltpu.sync_copy(idx.at[pl.ds(eb, cb)], idxb)
        pltpu.async_copy(table.at[idxb], rowsb, sem).wait()
        pltpu.sync_copy(rowsb, out.at[pl.ds(eb, cb)])
        return c

    lax.fori_loop(0, EPW // cb, chunk, 0)


@functools.lru_cache(maxsize=None)
def _make_gather(f, cb):
    return pl.kernel(
        functools.partial(_sc_gather_body, cb=cb),
        out_type=jax.ShapeDtypeStruct((E2P, f), jnp.float32),
        mesh=_mesh(),
        compiler_params=pltpu.CompilerParams(needs_layout_passes=False),
        scratch_types=[
            pltpu.VMEM((cb,), jnp.int32),
            pltpu.VMEM((cb, f), jnp.float32),
            pltpu.SemaphoreType.DMA,
        ],
    )


def _sc_den_body(ex, dst, out, dstb, exb, table):
    def zt(i, c):
        table[pl.ds(i * LN, LN)] = jnp.zeros((LN,), jnp.float32)
        return c

    lax.fori_loop(0, NP4 // LN, zt, 0)
    wid = _wid()
    CB = 432
    iota = lax.iota(jnp.int32, 16)

    def chunk(i, c):
        eb = wid * EPW + i * CB
        pltpu.sync_copy(dst.at[pl.ds(eb, CB)], dstb)
        pltpu.sync_copy(ex.at[pl.ds(eb * 8, CB * 8)], exb)

        def grp(g, c2):
            dstv = dstb[pl.ds(g * LN, LN)]
            rows = (g * LN + iota) * 8
            for h in range(4):
                ev = plsc.load_gather(exb, [rows + h])
                plsc.addupdate_scatter(table, [dstv * 4 + h], ev)
            return c2

        lax.fori_loop(0, CB // LN, grp, 0)
        return c

    lax.fori_loop(0, EPW // CB, chunk, 0)
    pltpu.sync_copy(table, out.at[wid])


@functools.lru_cache(maxsize=None)
def _sc_den_k():
    return pl.kernel(
        _sc_den_body,
        out_type=jax.ShapeDtypeStruct((NW, NP4), jnp.float32),
        mesh=_mesh(),
        compiler_params=pltpu.CompilerParams(needs_layout_passes=False),
        scratch_types=[
            pltpu.VMEM((432,), jnp.int32),
            pltpu.VMEM((432 * 8,), jnp.float32),
            pltpu.VMEM((NP4,), jnp.float32),
        ],
    )


def _sc_w_body(ex, inv, dst, out, dstb, exb, ab, invb):
    pltpu.sync_copy(inv, invb)
    wid = _wid()
    CB = 432
    iota = lax.iota(jnp.int32, 16)

    def chunk(i, c):
        eb = wid * EPW + i * CB
        pltpu.sync_copy(dst.at[pl.ds(eb, CB)], dstb)
        pltpu.sync_copy(ex.at[pl.ds(eb * 8, CB * 8)], exb)

        def grp(g, c2):
            dstv = dstb[pl.ds(g * LN, LN)]
            rows = (g * LN + iota) * 8
            for h in range(4):
                ev = plsc.load_gather(exb, [rows + h])
                iv = plsc.load_gather(invb, [dstv * 4 + h])
                plsc.store_scatter(ab, [rows + h], ev * iv)
            return c2

        lax.fori_loop(0, CB // LN, grp, 0)
        pltpu.sync_copy(ab, out.at[pl.ds(eb * 8, CB * 8)])
        return c

    lax.fori_loop(0, EPW // CB, chunk, 0)


@functools.lru_cache(maxsize=None)
def _sc_w_k():
    return pl.kernel(
        _sc_w_body,
        out_type=jax.ShapeDtypeStruct((E2P * 8,), jnp.float32),
        mesh=_mesh(),
        compiler_params=pltpu.CompilerParams(needs_layout_passes=False),
        scratch_types=[
            pltpu.VMEM((432,), jnp.int32),
            pltpu.VMEM((432 * 8,), jnp.float32),
            pltpu.VMEM((432 * 8,), jnp.float32),
            pltpu.VMEM((NP4,), jnp.float32),
        ],
    )


def _eap_body(epr, ea, e1w, e1b, e2w, e2b, out):
    h1 = jnp.maximum(epr[:] * e1w[:] + e1b[:], 0.0)
    ef = (h1 @ e2w[:] + e2b[:])[:, :4]
    b = epr.shape[0]
    out[:] = jnp.concatenate(
        [ea[:], ef, jnp.ones((b, 1), jnp.float32),
         jnp.zeros((b, 11), jnp.float32)], axis=1)


def _eap(epr2, ea, e1w, e1b, e2wp, e2bp):
    nb = E // 1000
    return pl.pallas_call(
        _eap_body,
        grid=(nb,),
        in_specs=[
            pl.BlockSpec((1000, 1), lambda i: (i, 0)),
            pl.BlockSpec((1000, 16), lambda i: (i, 0)),
            pl.BlockSpec((1, 32), lambda i: (0, 0)),
            pl.BlockSpec((1, 32), lambda i: (0, 0)),
            pl.BlockSpec((32, 128), lambda i: (0, 0)),
            pl.BlockSpec((1, 128), lambda i: (0, 0)),
        ],
        out_specs=pl.BlockSpec((1000, 32), lambda i: (i, 0)),
        out_shape=jax.ShapeDtypeStruct((E, 32), jnp.float32),
    )(epr2, ea, e1w, e1b, e2wp, e2bp)


def _mecomb_body(p, out):
    cnt = jnp.maximum(p[:, 20:21], 1.0)
    me = p[:] / cnt
    out[:] = jnp.concatenate(
        [me[:, :20], jnp.zeros((NP, 12), jnp.float32)], axis=1)


def _mecomb(psum):
    return pl.pallas_call(
        _mecomb_body,
        out_shape=jax.ShapeDtypeStruct((NP, 32), jnp.float32),
    )(psum)


def _lin_body(x, w, b, out):
    out[:] = jnp.dot(x[:], w[:], preferred_element_type=jnp.float32) + b[:]


def _lin(x, w, b):
    f = w.shape[1]
    return pl.pallas_call(
        _lin_body,
        grid=(NP // 1024,),
        in_specs=[
            pl.BlockSpec((1024, 128), lambda i: (i, 0)),
            pl.BlockSpec((128, f), lambda i: (0, 0)),
            pl.BlockSpec((1, f), lambda i: (0, 0)),
        ],
        out_specs=pl.BlockSpec((1024, f), lambda i: (i, 0)),
        out_shape=jax.ShapeDtypeStruct((NP, f), jnp.float32),
    )(x, w, b[None, :])


def _ee(ea2p, wep):
    f = wep.shape[1]
    return pl.pallas_call(
        _lin_body,
        grid=(E2P // 1024,),
        in_specs=[
            pl.BlockSpec((1024, 32), lambda i: (i, 0)),
            pl.BlockSpec((32, f), lambda i: (0, 0)),
            pl.BlockSpec((1, f), lambda i: (0, 0)),
        ],
        out_specs=pl.BlockSpec((1024, f), lambda i: (i, 0)),
        out_shape=jax.ShapeDtypeStruct((E2P, f), jnp.float32),
    )(ea2p, wep, jnp.zeros((1, f), jnp.float32))


def _alpha_body(msg, xrd, ee, attb, out):
    u = msg[:] + xrd[:] + ee[:]
    ker = jnp.where(u >= 0.0, u, 0.2 * u)
    p = ker * attb[:]
    f = p.shape[1]
    a4 = p.reshape(1024, 4, f // 4).sum(axis=-1)
    out[:] = jnp.concatenate([a4, a4], axis=1)


def _alpha(msg, xrd, ee, attb):
    f = attb.shape[1]
    return pl.pallas_call(
        _alpha_body,
        grid=(E2P // 1024,),
        in_specs=[
            pl.BlockSpec((1024, f), lambda i: (i, 0)),
            pl.BlockSpec((1024, f), lambda i: (i, 0)),
            pl.BlockSpec((1024, f), lambda i: (i, 0)),
            pl.BlockSpec((1, f), lambda i: (0, 0)),
        ],
        out_specs=pl.BlockSpec((1024, 8), lambda i: (i, 0)),
        out_shape=jax.ShapeDtypeStruct((E2P, 8), jnp.float32),
    )(msg, xrd, ee, attb)


def _gmax_body(alpha, out):
    i = pl.program_id(0)
    prev = jnp.where(i == 0, jnp.full((1, 1), -jnp.inf, jnp.float32), out[:])
    out[:] = jnp.maximum(prev, jnp.max(alpha[:]).reshape(1, 1))


def _gmax(alpha):
    return pl.pallas_call(
        _gmax_body,
        grid=(E2P // 1024,),
        in_specs=[pl.BlockSpec((1024, 8), lambda i: (i, 0))],
        out_specs=pl.BlockSpec((1, 1), lambda i: (0, 0)),
        out_shape=jax.ShapeDtypeStruct((1, 1), jnp.float32),
    )(alpha)


def _ex_body(alpha, gmax, out):
    out[:] = jnp.exp(alpha[:] - gmax[:])


def _ex(alpha, gmax):
    return pl.pallas_call(
        _ex_body,
        grid=(E2P // 1024,),
        in_specs=[
            pl.BlockSpec((1024, 8), lambda i: (i, 0)),
            pl.BlockSpec((1, 1), lambda i: (0, 0)),
        ],
        out_specs=pl.BlockSpec((1024, 8), lambda i: (i, 0)),
        out_shape=jax.ShapeDtypeStruct((E2P, 8), jnp.float32),
    )(alpha, gmax)


def _denred_body(parts, out, *, scale):
    den = jnp.sum(parts[:], axis=0, keepdims=True)
    out[:] = scale / (den + 1e-16)


def _denred(parts, scale):
    return pl.pallas_call(
        functools.partial(_denred_body, scale=scale),
        out_shape=jax.ShapeDtypeStruct((1, NP4), jnp.float32),
    )(parts)


def _scale_body(msg, a, out, *, f):
    av = a[:]
    if f == 128:
        iv = jnp.broadcast_to(
            av[:, :4][:, :, None], (1024, 4, 32)).reshape(1024, 128)
        out[:] = msg[:] * iv
    else:
        s = jnp.zeros((1024, 128), jnp.float32)
        for h in range(4):
            s = s + msg[:, h * 128:(h + 1) * 128] * av[:, h:h + 1]
        out[:] = s


def _scale(msg, a8, f):
    return pl.pallas_call(
        functools.partial(_scale_body, f=f),
        grid=(E2P // 1024,),
        in_specs=[
            pl.BlockSpec((1024, f), lambda i: (i, 0)),
            pl.BlockSpec((1024, 8), lambda i: (i, 0)),
        ],
        out_specs=pl.BlockSpec((1024, 128), lambda i: (i, 0)),
        out_shape=jax.ShapeDtypeStruct((E2P, 128), jnp.float32),
    )(msg, a8)


def _comb_body(agg, bias, out, *, elu):
    s = agg[:] + bias[:]
    if elu:
        s = jnp.where(s > 0.0, s, jnp.exp(jnp.minimum(s, 0.0)) - 1.0)
    out[:] = s


def _comb(agg, bias, elu):
    return pl.pallas_call(
        functools.partial(_comb_body, elu=elu),
        grid=(NP // 1024,),
        in_specs=[
            pl.BlockSpec((1024, 128), lambda i: (i, 0)),
            pl.BlockSpec((1, 128), lambda i: (0, 0)),
        ],
        out_specs=pl.BlockSpec((1024, 128), lambda i: (i, 0)),
        out_shape=jax.ShapeDtypeStruct((NP, 128), jnp.float32),
    )(agg, bias[None, :])


def _meanx_body(x, out):
    i = pl.program_id(0)
    rows = lax.broadcasted_iota(jnp.int32, (1024, 128), 0) + i * 1024
    xm = jnp.where(rows < N, x[:], 0.0)
    prev = jnp.where(i == 0, jnp.zeros((1, 128), jnp.float32), out[:])
    s = prev + jnp.sum(xm, axis=0, keepdims=True)
    out[:] = jnp.where(i == NP // 1024 - 1, s / N, s)


def _meanx(x):
    return pl.pallas_call(
        _meanx_body,
        grid=(NP // 1024,),
        in_specs=[pl.BlockSpec((1024, 128), lambda i: (i, 0))],
        out_specs=pl.BlockSpec((1, 128), lambda i: (0, 0)),
        out_shape=jax.ShapeDtypeStruct((1, 128), jnp.float32),
    )(x)


def _mlp_body(g, w1, b1, w2, b2, out):
    h = jnp.maximum(jnp.sum(g[:].T * w1[:], axis=0, keepdims=True) + b1[:], 0.0)
    s = jnp.sum(h.T * w2[:], axis=0, keepdims=True) + b2[:]
    out[:] = 1.0 / (1.0 + jnp.exp(-s))


def _mlp(g, w1, b1, w2, b2):
    return pl.pallas_call(
        _mlp_body,
        out_shape=jax.ShapeDtypeStruct((1, 1), jnp.float32),
    )(g, w1, b1[None, :], w2, b2[None, :])


def _layer(x, srcp, dstp, ea2p, p, f, elu):
    xl = _lin(x, p['Wl'], p['bl'])
    xr = _lin(x, p['Wr'], p['br'])
    wep = jnp.concatenate(
        [p['We'][:20], jnp.zeros((12, f), jnp.float32)], axis=0)
    ee = _ee(ea2p, wep)
    gather = _make_gather(128, 432) if f == 128 else _make_gather(512, 96)
    msg = gather(xl, srcp)
    xrd = gather(xr, dstp)
    attb = p['att'].reshape(1, f)
    alpha = _alpha(msg, xrd, ee, attb)
    ex = _ex(alpha, _gmax(alpha)).reshape(E2P * 8)
    dparts = _sc_den_k()(ex, dstp)
    inv = _denred(dparts, 1.0 if f == 128 else 0.25)
    a = _sc_w_k()(ex, inv.reshape(NP4), dstp).reshape(E2P, 8)
    scaled = _scale(msg, a, f)
    agg = jnp.zeros((NP, 128), jnp.float32).at[dstp].add(scaled)
    return _comb(agg, p['bias'], elu)


def kernel(node_features, edge_attr, epr_scores, question_emb, params, edge_index):
    src = edge_index[0].astype(jnp.int32)
    dst = edge_index[1].astype(jnp.int32)
    loop = jnp.arange(N, dtype=jnp.int32)
    pad = jnp.zeros((E2P - E2,), jnp.int32)
    srcp = jnp.concatenate([src, loop, pad])
    dstp = jnp.concatenate([dst, loop, pad + N])

    e2wp = jnp.concatenate(
        [params['e2_w'], jnp.zeros((32, 124), jnp.float32)], axis=1)
    e2bp = jnp.concatenate(
        [params['e2_b'], jnp.zeros((124,), jnp.float32)])[None, :]
    eap = _eap(epr_scores[:, None], edge_attr, params['e1_w'],
               params['e1_b'][None, :], e2wp, e2bp)
    msum = jnp.zeros((NP, 32), jnp.float32).at[dst].add(eap)
    eself = _mecomb(msum)
    ea2p = jnp.concatenate(
        [eap, eself[:N], jnp.zeros((E2P - E2, 32), jnp.float32)], axis=0)

    x = jnp.concatenate(
        [node_features, jnp.zeros((NP - N, 128), jnp.float32)], axis=0)
    x = _layer(x, srcp, dstp, ea2p, params['g0'], 128, True)
    x = _layer(x, srcp, dstp, ea2p, params['g1'], 128, True)
    x = _layer(x, srcp, dstp, ea2p, params['g2'], 512, False)

    graph_repr = _meanx(x)
    path_score = _mlp(graph_repr, params['s1_w'], params['s1_b'],
                      params['s2_w'], params['s2_b'])
    return path_score.reshape(1), x[:N], graph_repr.reshape(128)
